# baseline (speedup 1.0000x reference)
import functools

import jax
import jax.numpy as jnp
from jax import lax
from jax.experimental import pallas as pl
from jax.experimental.pallas import tpu as pltpu
from jax.experimental.pallas import tpu_sc as plsc

_CHUNK = 128


@functools.lru_cache(maxsize=None)
def _make_gather(V, D, B):
    info = plsc.get_sparse_core_info()
    NC, NS = info.num_cores, info.num_subcores
    NW = NC * NS
    assert B % NW == 0
    b_per_w = B // NW
    ch = min(_CHUNK, b_per_w)
    assert b_per_w % ch == 0
    n_chunks = b_per_w // ch
    mesh = plsc.VectorSubcoreMesh(core_axis_name="c", subcore_axis_name="s")

    @functools.partial(
        pl.kernel,
        mesh=mesh,
        compiler_params=pltpu.CompilerParams(use_tc_tiling_on_sc=False),
        out_type=jax.ShapeDtypeStruct((B, D), jnp.float32),
        scratch_types=[
            pltpu.VMEM((n_chunks, ch), jnp.int32),
            pltpu.VMEM((b_per_w, D), jnp.float32),
            pltpu.SemaphoreType.DMA,
        ],
    )
    def gather_kernel(table_hbm, idx_hbm, out_hbm, idx_v, rows_v, sem):
        wid = lax.axis_index("s") * NC + lax.axis_index("c")
        base = wid * b_per_w
        pltpu.sync_copy(idx_hbm.at[wid], idx_v)
        copies = [
            pltpu.async_copy(
                table_hbm.at[idx_v.at[j]], rows_v.at[pl.ds(j * ch, ch)], sem
            )
            for j in range(n_chunks)
        ]
        for c in copies:
            c.wait()
        pltpu.sync_copy(rows_v, out_hbm.at[pl.ds(base, b_per_w)])

    def run(table, idx):
        return gather_kernel(table, idx.reshape(NW, n_chunks, ch))

    return run, NW, n_chunks, ch


def kernel(embedding_table, labels, train):
    V, D = embedding_table.shape
    (B,) = labels.shape
    run, _, _, _ = _make_gather(V, D, B)
    return run(embedding_table, labels.astype(jnp.int32))


# speedup vs baseline: 1.7307x; 1.7307x over previous
import functools

import jax
import jax.numpy as jnp
from jax import lax
from jax.experimental import pallas as pl
from jax.experimental.pallas import tpu as pltpu
from jax.experimental.pallas import tpu_sc as plsc

_LANES = 16
_GROUPS_IN_FLIGHT = 8


@functools.lru_cache(maxsize=None)
def _make_gather(V, D, B):
    info = plsc.get_sparse_core_info()
    NC, NS = info.num_cores, info.num_subcores
    NW = NC * NS
    assert B % (NW * _LANES) == 0
    b_per_w = B // NW
    n_groups = b_per_w // _LANES
    mesh = plsc.VectorSubcoreMesh(core_axis_name="c", subcore_axis_name="s")

    @functools.partial(
        pl.kernel,
        mesh=mesh,
        compiler_params=pltpu.CompilerParams(needs_layout_passes=False),
        out_type=jax.ShapeDtypeStruct((B, D), jnp.float32),
        scratch_types=[
            pltpu.VMEM((b_per_w,), jnp.int32),
            pltpu.VMEM((b_per_w, D), jnp.float32),
            pltpu.SemaphoreType.DMA,
        ],
    )
    def gather_kernel(table_hbm, idx_hbm, out_hbm, idx_v, rows_v, sem):
        wid = lax.axis_index("s") * NC + lax.axis_index("c")
        base = wid * b_per_w
        pltpu.sync_copy(idx_hbm.at[pl.ds(base, b_per_w)], idx_v)
        lane = lax.iota(jnp.int32, _LANES)
        lag = min(_GROUPS_IN_FLIGHT, n_groups)

        def drain_one_group():
            pltpu.make_async_copy(
                table_hbm.at[pl.ds(0, _LANES)],
                rows_v.at[pl.ds(0, _LANES)],
                sem,
            ).wait()

        def group(g, _):
            vec = idx_v[pl.ds(g * _LANES, _LANES)]

            def fire(j, _):
                r = jnp.max(jnp.where(lane == j, vec, 0))
                pltpu.async_copy(
                    table_hbm.at[pl.ds(r, 1)],
                    rows_v.at[pl.ds(g * _LANES + j, 1)],
                    sem,
                )
                return 0

            lax.fori_loop(0, _LANES, fire, 0)
            @pl.when(g >= lag)
            def _():
                drain_one_group()

            return 0

        lax.fori_loop(0, n_groups, group, 0)

        def drain(g, _):
            drain_one_group()
            return 0

        lax.fori_loop(0, lag, drain, 0)
        pltpu.sync_copy(rows_v, out_hbm.at[pl.ds(base, b_per_w)])

    return gather_kernel


def kernel(embedding_table, labels, train):
    V, D = embedding_table.shape
    (B,) = labels.shape
    gather = _make_gather(V, D, B)
    return gather(embedding_table, labels.astype(jnp.int32))


# speedup vs baseline: 1.7341x; 1.0019x over previous
import functools

import jax
import jax.numpy as jnp
from jax import lax
from jax.experimental import pallas as pl
from jax.experimental.pallas import tpu as pltpu
from jax.experimental.pallas import tpu_sc as plsc

_LANES = 16
_GROUPS_IN_FLIGHT = 8


@functools.lru_cache(maxsize=None)
def _make_gather(V, D, B):
    info = plsc.get_sparse_core_info()
    NC, NS = info.num_cores, info.num_subcores
    NW = NC * NS
    assert B % (NW * _LANES) == 0
    b_per_w = B // NW
    n_groups = b_per_w // _LANES
    mesh = plsc.VectorSubcoreMesh(core_axis_name="c", subcore_axis_name="s")

    @functools.partial(
        pl.kernel,
        mesh=mesh,
        out_type=jax.ShapeDtypeStruct((B, D), jnp.float32),
        scratch_types=[
            pltpu.VMEM((b_per_w,), jnp.int32),
            pltpu.VMEM((b_per_w, D), jnp.float32),
            pltpu.SemaphoreType.DMA,
        ],
    )
    def gather_kernel(table_hbm, idx_hbm, out_hbm, idx_v, rows_v, sem):
        wid = lax.axis_index("s") * NC + lax.axis_index("c")
        base = wid * b_per_w
        pltpu.sync_copy(idx_hbm.at[pl.ds(base, b_per_w)], idx_v)
        lag = min(_GROUPS_IN_FLIGHT, n_groups)

        def drain_one_group():
            pltpu.make_async_copy(
                table_hbm.at[pl.ds(0, _LANES)],
                rows_v.at[pl.ds(0, _LANES)],
                sem,
            ).wait()

        def group(g, _):
            vec = idx_v[pl.ds(g * _LANES, _LANES)]
            for j in range(_LANES):
                r = vec[j]
                pltpu.async_copy(
                    table_hbm.at[pl.ds(r, 1)],
                    rows_v.at[pl.ds(g * _LANES + j, 1)],
                    sem,
                )
            @pl.when(g >= lag)
            def _():
                drain_one_group()

            return 0

        lax.fori_loop(0, n_groups, group, 0)

        def drain(g, _):
            drain_one_group()
            return 0

        lax.fori_loop(0, lag, drain, 0)
        pltpu.sync_copy(rows_v, out_hbm.at[pl.ds(base, b_per_w)])

    return gather_kernel


def kernel(embedding_table, labels, train):
    V, D = embedding_table.shape
    (B,) = labels.shape
    gather = _make_gather(V, D, B)
    return gather(embedding_table, labels.astype(jnp.int32))


# speedup vs baseline: 3.2158x; 1.8545x over previous
import functools

import jax
import jax.numpy as jnp
from jax import lax
from jax.experimental import pallas as pl
from jax.experimental.pallas import tpu as pltpu
from jax.experimental.pallas import tpu_sc as plsc

_LANES = 16
_CH = 512


@functools.lru_cache(maxsize=None)
def _make_gather(V, D, B):
    info = plsc.get_sparse_core_info()
    NC, NS = info.num_cores, info.num_subcores
    NW = NC * NS
    assert B % (NW * _LANES) == 0 and D % _LANES == 0
    bpw = B // NW
    n_groups = bpw // _LANES
    v_pad = pl.cdiv(V, 128) * 128
    lo_max = ((v_pad - _CH) // 128) * 128
    mesh = plsc.VectorSubcoreMesh(core_axis_name="c", subcore_axis_name="s")

    @functools.partial(
        pl.kernel,
        mesh=mesh,
        compiler_params=pltpu.CompilerParams(needs_layout_passes=False),
        out_type=jax.ShapeDtypeStruct((B, D), jnp.float32),
        scratch_types=[
            pltpu.VMEM((bpw,), jnp.int32),
            pltpu.VMEM((bpw,), jnp.int32),
            pltpu.VMEM((D, _CH), jnp.float32),
            pltpu.VMEM((_LANES, 1, D), jnp.float32),
            pltpu.SemaphoreType.DMA,
        ],
    )
    def gather_kernel(tab_hbm, r_hbm, s_hbm, out_hbm, r_v, s_v, chunk_v,
                      colbuf_v, sem):
        wid = lax.axis_index("s") * NC + lax.axis_index("c")
        e0 = wid * bpw
        pltpu.sync_copy(r_hbm.at[pl.ds(e0, bpw)], r_v)
        pltpu.sync_copy(s_hbm.at[pl.ds(e0, bpw)], s_v)
        d_iota = lax.iota(jnp.int32, _LANES)

        def group(g, cur_lo):
            vec_r = r_v[pl.ds(g * _LANES, _LANES)]
            vec_s = s_v[pl.ds(g * _LANES, _LANES)]
            for j in range(_LANES):
                r_j = vec_r[j]

                def load_new(r_j=r_j):
                    nl = jnp.minimum((r_j // _CH) * _CH, lo_max)
                    nl = pl.multiple_of(nl, 128)
                    pltpu.sync_copy(tab_hbm.at[:, pl.ds(nl, _CH)], chunk_v)
                    return nl

                cur_lo = lax.cond(
                    r_j >= cur_lo + _CH, load_new, lambda c=cur_lo: c
                )
                col = jnp.broadcast_to(r_j - cur_lo, (_LANES,))
                for i in range(D // _LANES):
                    g_vals = plsc.load_gather(
                        chunk_v, [d_iota + _LANES * i, col]
                    )
                    colbuf_v[j, 0, pl.ds(_LANES * i, _LANES)] = g_vals
                pltpu.async_copy(
                    colbuf_v.at[j], out_hbm.at[pl.ds(vec_s[j], 1)], sem
                )
            for j in range(_LANES):
                pltpu.make_async_copy(
                    colbuf_v.at[j], out_hbm.at[pl.ds(0, 1)], sem
                ).wait()
            return cur_lo

        lax.fori_loop(0, n_groups, group, jnp.int32(-_CH))

    return gather_kernel


def kernel(embedding_table, labels, train):
    V, D = embedding_table.shape
    (B,) = labels.shape
    idx = labels.astype(jnp.int32)
    slots = lax.iota(jnp.int32, B)
    sorted_r, order = lax.sort((idx, slots), num_keys=1)
    gather = _make_gather(V, D, B)
    return gather(embedding_table.T, sorted_r, order)


# speedup vs baseline: 3.7146x; 1.1551x over previous
import functools

import jax
import jax.numpy as jnp
from jax import lax
from jax.experimental import pallas as pl
from jax.experimental.pallas import tpu as pltpu
from jax.experimental.pallas import tpu_sc as plsc

_LANES = 16
_CH = 1536


@functools.lru_cache(maxsize=None)
def _make_gather(V, D, B):
    info = plsc.get_sparse_core_info()
    NC, NS = info.num_cores, info.num_subcores
    NW = NC * NS
    assert B % (NW * _LANES) == 0 and D % _LANES == 0
    bpw = B // NW
    n_groups = bpw // _LANES
    v_pad = pl.cdiv(V, 128) * 128
    lo_max = ((v_pad - _CH) // 128) * 128
    mesh = plsc.VectorSubcoreMesh(core_axis_name="c", subcore_axis_name="s")

    @functools.partial(
        pl.kernel,
        mesh=mesh,
        compiler_params=pltpu.CompilerParams(needs_layout_passes=False),
        out_type=jax.ShapeDtypeStruct((B, D), jnp.float32),
        scratch_types=[
            pltpu.VMEM((bpw,), jnp.int32),
            pltpu.VMEM((bpw,), jnp.int32),
            pltpu.VMEM((D, _CH), jnp.float32),
            pltpu.VMEM((_LANES, 1, D), jnp.float32),
            pltpu.SemaphoreType.DMA,
        ],
    )
    def gather_kernel(tab_hbm, r_hbm, s_hbm, out_hbm, r_v, s_v, chunk_v,
                      colbuf_v, sem):
        wid = lax.axis_index("s") * NC + lax.axis_index("c")
        e0 = wid * bpw
        pltpu.sync_copy(r_hbm.at[pl.ds(e0, bpw)], r_v)
        pltpu.sync_copy(s_hbm.at[pl.ds(e0, bpw)], s_v)
        d_iota = lax.iota(jnp.int32, _LANES)

        def group(g, cur_lo):
            vec_r = r_v[pl.ds(g * _LANES, _LANES)]
            vec_s = s_v[pl.ds(g * _LANES, _LANES)]
            for j in range(_LANES):
                r_j = vec_r[j]

                def load_new(r_j=r_j):
                    nl = jnp.minimum((r_j // _CH) * _CH, lo_max)
                    nl = pl.multiple_of(nl, 128)
                    pltpu.sync_copy(tab_hbm.at[:, pl.ds(nl, _CH)], chunk_v)
                    return nl

                cur_lo = lax.cond(
                    r_j >= cur_lo + _CH, load_new, lambda c=cur_lo: c
                )
                col = jnp.broadcast_to(r_j - cur_lo, (_LANES,))
                for i in range(D // _LANES):
                    g_vals = plsc.load_gather(
                        chunk_v, [d_iota + _LANES * i, col]
                    )
                    colbuf_v[j, 0, pl.ds(_LANES * i, _LANES)] = g_vals
                pltpu.async_copy(
                    colbuf_v.at[j], out_hbm.at[pl.ds(vec_s[j], 1)], sem
                )
            for j in range(_LANES):
                pltpu.make_async_copy(
                    colbuf_v.at[j], out_hbm.at[pl.ds(0, 1)], sem
                ).wait()
            return cur_lo

        lax.fori_loop(0, n_groups, group, jnp.int32(-_CH))

    return gather_kernel


def kernel(embedding_table, labels, train):
    V, D = embedding_table.shape
    (B,) = labels.shape
    idx = labels.astype(jnp.int32)
    slots = lax.iota(jnp.int32, B)
    sorted_r, order = lax.sort((idx, slots), num_keys=1)
    gather = _make_gather(V, D, B)
    return gather(embedding_table.T, sorted_r, order)


# speedup vs baseline: 4.0615x; 1.0934x over previous
import functools

import jax
import jax.numpy as jnp
from jax import lax
from jax.experimental import pallas as pl
from jax.experimental.pallas import tpu as pltpu
from jax.experimental.pallas import tpu_sc as plsc

_LANES = 16
_CH = 768


@functools.lru_cache(maxsize=None)
def _make_gather(V, D, B):
    info = plsc.get_sparse_core_info()
    NC, NS = info.num_cores, info.num_subcores
    NW = NC * NS
    assert B % (NW * _LANES) == 0 and D % _LANES == 0
    bpw = B // NW
    n_groups = bpw // _LANES
    v_pad = pl.cdiv(V, 128) * 128
    lo_max = ((v_pad - _CH) // 128) * 128
    mesh = plsc.VectorSubcoreMesh(core_axis_name="c", subcore_axis_name="s")

    @functools.partial(
        pl.kernel,
        mesh=mesh,
        compiler_params=pltpu.CompilerParams(needs_layout_passes=False),
        out_type=jax.ShapeDtypeStruct((B, D), jnp.float32),
        scratch_types=[
            pltpu.VMEM((bpw,), jnp.int32),
            pltpu.VMEM((bpw,), jnp.int32),
            pltpu.VMEM((D, _CH), jnp.float32),
            pltpu.VMEM((D, _CH), jnp.float32),
            pltpu.VMEM((_LANES, 1, D), jnp.float32),
            pltpu.SemaphoreType.DMA,
            pltpu.SemaphoreType.DMA,
        ],
    )
    def gather_kernel(tab_hbm, r_hbm, s_hbm, out_hbm, r_v, s_v, chunk0_v,
                      chunk1_v, colbuf_v, sem, sem_pf):
        wid = lax.axis_index("s") * NC + lax.axis_index("c")
        e0 = wid * bpw
        pltpu.sync_copy(r_hbm.at[pl.ds(e0, bpw)], r_v)
        pltpu.sync_copy(s_hbm.at[pl.ds(e0, bpw)], s_v)
        d_iota = lax.iota(jnp.int32, _LANES)
        chunks = (chunk0_v, chunk1_v)

        def clamp(lo):
            return pl.multiple_of(jnp.minimum(lo, lo_max), 128)

        def pf_start(lo, buf):
            pltpu.async_copy(tab_hbm.at[:, pl.ds(clamp(lo), _CH)], buf, sem_pf)

        def pf_wait(buf):
            pltpu.make_async_copy(
                tab_hbm.at[:, pl.ds(0, _CH)], buf, sem_pf
            ).wait()

        first_lo = clamp((r_v[pl.ds(0, _LANES)][0] // _CH) * _CH)
        pltpu.sync_copy(tab_hbm.at[:, pl.ds(first_lo, _CH)], chunk0_v)
        pf_start(first_lo + _CH, chunk1_v)

        def group(g, carry):
            cur_lo, parity = carry
            vec_r = r_v[pl.ds(g * _LANES, _LANES)]
            vec_s = s_v[pl.ds(g * _LANES, _LANES)]
            for j in range(_LANES):
                r_j = vec_r[j]

                def seq_adv(cur_lo=cur_lo, parity=parity):
                    nl = clamp(cur_lo + _CH)
                    for q in (0, 1):
                        @pl.when(parity == q)
                        def _(q=q):
                            pf_wait(chunks[1 - q])
                            pf_start(nl + _CH, chunks[q])
                    return nl, 1 - parity

                def jump(cur_lo=cur_lo, parity=parity, r_j=r_j):
                    nl = clamp((r_j // _CH) * _CH)
                    for q in (0, 1):
                        @pl.when(parity == q)
                        def _(q=q):
                            pf_wait(chunks[1 - q])
                            pltpu.sync_copy(
                                tab_hbm.at[:, pl.ds(nl, _CH)],
                                chunks[q],
                            )
                            pf_start(nl + _CH, chunks[1 - q])
                    return nl, parity

                def advance(cur_lo=cur_lo, parity=parity, r_j=r_j):
                    return lax.cond(
                        r_j < cur_lo + 2 * _CH, seq_adv, jump
                    )

                cur_lo, parity = lax.cond(
                    r_j >= cur_lo + _CH,
                    advance,
                    lambda c=cur_lo, q=parity: (c, q),
                )
                col = jnp.broadcast_to(r_j - cur_lo, (_LANES,))
                for q in (0, 1):
                    @pl.when(parity == q)
                    def _(q=q, col=col, j=j):
                        for i in range(D // _LANES):
                            g_vals = plsc.load_gather(
                                chunks[q], [d_iota + _LANES * i, col]
                            )
                            colbuf_v[j, 0, pl.ds(_LANES * i, _LANES)] = g_vals
                pltpu.async_copy(
                    colbuf_v.at[j], out_hbm.at[pl.ds(vec_s[j], 1)], sem
                )
            for j in range(_LANES):
                pltpu.make_async_copy(
                    colbuf_v.at[j], out_hbm.at[pl.ds(0, 1)], sem
                ).wait()
            return cur_lo, parity

        _, parity = lax.fori_loop(
            0, n_groups, group, (first_lo, jnp.int32(0))
        )
        for q in (0, 1):
            @pl.when(parity == q)
            def _(q=q):
                pf_wait(chunks[1 - q])

    return gather_kernel


def kernel(embedding_table, labels, train):
    V, D = embedding_table.shape
    (B,) = labels.shape
    idx = labels.astype(jnp.int32)
    slots = lax.iota(jnp.int32, B)
    sorted_r, order = lax.sort((idx, slots), num_keys=1)
    gather = _make_gather(V, D, B)
    return gather(embedding_table.T, sorted_r, order)
